# baseline (speedup 1.0000x reference)
import functools

import jax
import jax.numpy as jnp
from jax import lax
from jax.experimental import pallas as pl
from jax.experimental.pallas import tpu as pltpu
from jax.experimental.pallas import tpu_sc as plsc

NUM_EMBEDDINGS = 1000000
EMBEDDING_DIM = 64

_info = plsc.get_sparse_core_info()
NC, NS = _info.num_cores, _info.num_subcores
NW = NC * NS

B_TOTAL = 16384 * 50
CHUNK = 512
N_CHUNKS = B_TOTAL // (NW * CHUNK)
B_PER_W = N_CHUNKS * CHUNK
NBUF = 3


def _gather_kernel(idx_hbm, table_hbm, out_hbm,
                   idx_v, rows_0, rows_1, rows_2,
                   sg0, sg1, sg2, ss0, ss1, ss2):
    wid = lax.axis_index("s") * NC + lax.axis_index("c")
    base = wid * B_PER_W
    rows = (rows_0, rows_1, rows_2)
    sg = (sg0, sg1, sg2)
    ss = (ss0, ss1, ss2)

    pltpu.sync_copy(idx_hbm.at[wid], idx_v)

    pltpu.async_copy(table_hbm.at[idx_v.at[0]], rows[0], sg[0])
    pltpu.async_copy(table_hbm.at[idx_v.at[1]], rows[1], sg[1])

    def step(j, k):
        kp = (k - 1) % NBUF
        pltpu.make_async_copy(table_hbm.at[idx_v.at[j]], rows[k], sg[k]).wait()
        pltpu.async_copy(rows[k], out_hbm.at[pl.ds(base + j * CHUNK, CHUNK)],
                         ss[k])

        @pl.when(j >= 1)
        def _drain_prev_store():
            pltpu.make_async_copy(
                rows[kp], out_hbm.at[pl.ds(base + (j - 1) * CHUNK, CHUNK)],
                ss[kp]).wait()

        @pl.when(j + 2 < N_CHUNKS)
        def _gather_ahead():
            pltpu.async_copy(table_hbm.at[idx_v.at[j + 2]], rows[kp], sg[kp])

    def body(j, carry):
        m = lax.rem(j, NBUF)
        for k in range(NBUF):
            @pl.when(m == k)
            def _(k=k):
                step(j, k)
        return carry

    lax.fori_loop(0, N_CHUNKS, body, 0, unroll=False)

    k_last = (N_CHUNKS - 1) % NBUF
    pltpu.make_async_copy(
        rows[k_last],
        out_hbm.at[pl.ds(base + (N_CHUNKS - 1) * CHUNK, CHUNK)],
        ss[k_last]).wait()


@jax.jit
def _embedding_lookup(x, weight):
    idx = x.reshape(NW, N_CHUNKS, CHUNK).astype(jnp.int32)
    mesh = plsc.VectorSubcoreMesh(core_axis_name="c", subcore_axis_name="s")
    out = pl.kernel(
        _gather_kernel,
        mesh=mesh,
        out_type=jax.ShapeDtypeStruct((B_TOTAL, EMBEDDING_DIM), jnp.float32),
        scratch_types=[
            pltpu.VMEM((N_CHUNKS, CHUNK), jnp.int32),
            pltpu.VMEM((CHUNK, EMBEDDING_DIM), jnp.float32),
            pltpu.VMEM((CHUNK, EMBEDDING_DIM), jnp.float32),
            pltpu.VMEM((CHUNK, EMBEDDING_DIM), jnp.float32),
            pltpu.SemaphoreType.DMA,
            pltpu.SemaphoreType.DMA,
            pltpu.SemaphoreType.DMA,
            pltpu.SemaphoreType.DMA,
            pltpu.SemaphoreType.DMA,
            pltpu.SemaphoreType.DMA,
        ],
        compiler_params=pltpu.CompilerParams(use_tc_tiling_on_sc=False),
    )(idx, weight)
    return out.reshape(x.shape + (EMBEDDING_DIM,))


def kernel(x, weight):
    return _embedding_lookup(x, weight)
